# baseline (speedup 1.0000x reference)
import jax
import jax.numpy as jnp
from jax.experimental import pallas as pl
from jax.experimental.pallas import tpu as pltpu

ROWS = 4
A = 15


def _rpn_head_kernel(x_ref, xup_ref, xdn_ref, ws_ref, bs_ref,
                     wc_ref, bc_ref, wd_ref, bd_ref,
                     cls_ref, probs_ref, delta_ref,
                     cls_buf, probs_buf, delta_buf, sem):
    b = pl.program_id(0)
    i = pl.program_id(1)
    nblk = pl.num_programs(1)
    t = b * nblk + i
    slot = jax.lax.rem(t, 2)
    r = ROWS
    m = r * 128
    blk = m * A

    def _copies(s, bb, ii):
        return [
            pltpu.make_async_copy(
                cls_buf.at[s],
                cls_ref.at[bb, pl.ds(ii * blk, blk), :], sem.at[s, 0]),
            pltpu.make_async_copy(
                probs_buf.at[s],
                probs_ref.at[bb, pl.ds(ii * blk, blk), :], sem.at[s, 1]),
            pltpu.make_async_copy(
                delta_buf.at[s],
                delta_ref.at[bb, pl.ds(ii * blk, blk), :], sem.at[s, 2]),
        ]

    @pl.when(t >= 2)
    def _wait_prev():
        for c in _copies(slot, b, i):
            c.wait()

    x = x_ref[0].astype(jnp.bfloat16)
    up = xup_ref[0].astype(jnp.bfloat16)
    dn = xdn_ref[0].astype(jnp.bfloat16)
    zero_row = jnp.zeros_like(up)
    up = jnp.where(i == 0, zero_row, up)
    dn = jnp.where(i == nblk - 1, zero_row, dn)
    xg = jnp.concatenate([up, x, dn], axis=0)
    xg = jnp.pad(xg, ((0, 0), (1, 1), (0, 0)))

    acc = jnp.zeros((m, 512), jnp.float32)
    for dy in range(3):
        for dx in range(3):
            patch = xg[dy:dy + r, dx:dx + 128, :].reshape(m, 256)
            acc += jnp.dot(patch, ws_ref[dy, dx],
                           preferred_element_type=jnp.float32)

    shared = jnp.clip(acc + bs_ref[...], 0.0, 6.0)
    sh = shared.astype(jnp.bfloat16)

    xc = jnp.dot(sh, wc_ref[...], preferred_element_type=jnp.float32)
    xc = xc + bc_ref[...]
    xd = jnp.dot(sh, wd_ref[...], preferred_element_type=jnp.float32)
    xd = xd + bd_ref[...]

    lane = jax.lax.broadcasted_iota(jnp.int32, (m, 30), 1)
    partner = jnp.where(lane % 2 == 0,
                        jnp.roll(xc, -1, axis=1),
                        jnp.roll(xc, 1, axis=1))
    probs = jax.nn.sigmoid(xc - partner)

    for k in range(A):
        rows = pl.Slice(k, m, A)
        cls_buf[slot, rows, :] = xc[:, 2 * k:2 * k + 2]
        probs_buf[slot, rows, :] = probs[:, 2 * k:2 * k + 2]
        delta_buf[slot, rows, :] = xd[:, 4 * k:4 * k + 4]

    for c in _copies(slot, b, i):
        c.start()

    total = pl.num_programs(0) * nblk

    @pl.when(t == total - 1)
    def _drain():
        for c in _copies(slot, b, i):
            c.wait()
        for c in _copies(1 - slot, b, i):
            c.wait()


def kernel(inputs, W_shared, b_shared, W_cls, b_cls, W_delta, b_delta):
    x = inputs[0]
    B, H, W, C = x.shape
    r = ROWS
    nblk = H // r
    n_anchor = H * W * A

    ws = W_shared.astype(jnp.bfloat16)
    wc = W_cls[0, 0].astype(jnp.bfloat16)
    wd = W_delta[0, 0].astype(jnp.bfloat16)
    bs = b_shared.reshape(1, -1)
    bc = b_cls.reshape(1, -1)
    bd = b_delta.reshape(1, -1)

    blk = r * W * A
    grid = (B, nblk)
    in_specs = [
        pl.BlockSpec((1, r, W, C), lambda b, i: (b, i, 0, 0)),
        pl.BlockSpec((1, 1, W, C),
                     lambda b, i: (b, jnp.maximum(i * ROWS - 1, 0), 0, 0)),
        pl.BlockSpec((1, 1, W, C),
                     lambda b, i: (b, jnp.minimum(i * ROWS + ROWS, 127), 0, 0)),
        pl.BlockSpec((3, 3, C, 512), lambda b, i: (0, 0, 0, 0)),
        pl.BlockSpec((1, 512), lambda b, i: (0, 0)),
        pl.BlockSpec((512, 30), lambda b, i: (0, 0)),
        pl.BlockSpec((1, 30), lambda b, i: (0, 0)),
        pl.BlockSpec((512, 60), lambda b, i: (0, 0)),
        pl.BlockSpec((1, 60), lambda b, i: (0, 0)),
    ]
    out_specs = [
        pl.BlockSpec(memory_space=pl.ANY),
        pl.BlockSpec(memory_space=pl.ANY),
        pl.BlockSpec(memory_space=pl.ANY),
    ]
    out_shapes = [
        jax.ShapeDtypeStruct((B, n_anchor, 2), jnp.float32),
        jax.ShapeDtypeStruct((B, n_anchor, 2), jnp.float32),
        jax.ShapeDtypeStruct((B, n_anchor, 4), jnp.float32),
    ]
    scratch_shapes = [
        pltpu.VMEM((2, blk, 2), jnp.float32),
        pltpu.VMEM((2, blk, 2), jnp.float32),
        pltpu.VMEM((2, blk, 4), jnp.float32),
        pltpu.SemaphoreType.DMA((2, 3)),
    ]
    logits, probs, deltas = pl.pallas_call(
        _rpn_head_kernel,
        grid=grid,
        in_specs=in_specs,
        out_specs=out_specs,
        out_shape=out_shapes,
        scratch_shapes=scratch_shapes,
    )(x, x, x, ws, bs, wc, bc, wd, bd)
    return (logits, probs, deltas)


# speedup vs baseline: 1.0044x; 1.0044x over previous
import jax
import jax.numpy as jnp
from jax.experimental import pallas as pl

ROWS = 4
A = 15


def _rpn_head_kernel(x_ref, xup_ref, xdn_ref, ws_ref, bs_ref,
                     wc_ref, bc_ref, wd_ref, bd_ref,
                     cls_ref, probs_ref, delta_ref):
    i = pl.program_id(1)
    nblk = pl.num_programs(1)
    r = ROWS
    x = x_ref[0].astype(jnp.bfloat16)
    up = xup_ref[0].astype(jnp.bfloat16)
    dn = xdn_ref[0].astype(jnp.bfloat16)
    zero_row = jnp.zeros_like(up)
    up = jnp.where(i == 0, zero_row, up)
    dn = jnp.where(i == nblk - 1, zero_row, dn)
    xg = jnp.concatenate([up, x, dn], axis=0)
    xg = jnp.pad(xg, ((0, 0), (1, 1), (0, 0)))

    m = r * 128
    acc = jnp.zeros((m, 512), jnp.float32)
    for dy in range(3):
        for dx in range(3):
            patch = xg[dy:dy + r, dx:dx + 128, :].reshape(m, 256)
            acc += jnp.dot(patch, ws_ref[dy, dx],
                           preferred_element_type=jnp.float32)

    shared = jnp.clip(acc + bs_ref[...], 0.0, 6.0)
    sh = shared.astype(jnp.bfloat16)

    xc = jnp.dot(sh, wc_ref[...], preferred_element_type=jnp.float32)
    xc = xc + bc_ref[...]
    xd = jnp.dot(sh, wd_ref[...], preferred_element_type=jnp.float32)
    xd = xd + bd_ref[...]

    lane = jax.lax.broadcasted_iota(jnp.int32, (m, 30), 1)
    partner = jnp.where(lane % 2 == 0,
                        jnp.roll(xc, -1, axis=1),
                        jnp.roll(xc, 1, axis=1))
    probs = jax.nn.sigmoid(xc - partner)

    for k in range(A):
        rows = pl.Slice(k, m, A)
        cls_ref[0, rows, :] = xc[:, 2 * k:2 * k + 2]
        probs_ref[0, rows, :] = probs[:, 2 * k:2 * k + 2]
        delta_ref[0, rows, :] = xd[:, 4 * k:4 * k + 4]


def kernel(inputs, W_shared, b_shared, W_cls, b_cls, W_delta, b_delta):
    x = inputs[0]
    B, H, W, C = x.shape
    r = ROWS
    nblk = H // r
    n_anchor = H * W * A

    ws = W_shared.astype(jnp.bfloat16)
    wc = W_cls[0, 0].astype(jnp.bfloat16)
    wd = W_delta[0, 0].astype(jnp.bfloat16)
    bs = b_shared.reshape(1, -1)
    bc = b_cls.reshape(1, -1)
    bd = b_delta.reshape(1, -1)

    blk = r * W * A
    grid = (B, nblk)
    in_specs = [
        pl.BlockSpec((1, r, W, C), lambda b, i: (b, i, 0, 0)),
        pl.BlockSpec((1, 1, W, C),
                     lambda b, i: (b, jnp.maximum(i * ROWS - 1, 0), 0, 0)),
        pl.BlockSpec((1, 1, W, C),
                     lambda b, i: (b, jnp.minimum(i * ROWS + ROWS, 127), 0, 0)),
        pl.BlockSpec((3, 3, C, 512), lambda b, i: (0, 0, 0, 0)),
        pl.BlockSpec((1, 512), lambda b, i: (0, 0)),
        pl.BlockSpec((512, 30), lambda b, i: (0, 0)),
        pl.BlockSpec((1, 30), lambda b, i: (0, 0)),
        pl.BlockSpec((512, 60), lambda b, i: (0, 0)),
        pl.BlockSpec((1, 60), lambda b, i: (0, 0)),
    ]
    out_specs = [
        pl.BlockSpec((1, blk, 2), lambda b, i: (b, i, 0)),
        pl.BlockSpec((1, blk, 2), lambda b, i: (b, i, 0)),
        pl.BlockSpec((1, blk, 4), lambda b, i: (b, i, 0)),
    ]
    out_shapes = [
        jax.ShapeDtypeStruct((B, n_anchor, 2), jnp.float32),
        jax.ShapeDtypeStruct((B, n_anchor, 2), jnp.float32),
        jax.ShapeDtypeStruct((B, n_anchor, 4), jnp.float32),
    ]
    logits, probs, deltas = pl.pallas_call(
        _rpn_head_kernel,
        grid=grid,
        in_specs=in_specs,
        out_specs=out_specs,
        out_shape=out_shapes,
    )(x, x, x, ws, bs, wc, bc, wd, bd)
    return (logits, probs, deltas)
